# baseline (speedup 1.0000x reference)
import functools
import math

import jax
import jax.numpy as jnp
import numpy as np
from jax.experimental import pallas as pl

_ANCHOR_SCALES = np.array([8.0, 16.0, 32.0])
_ANCHOR_RATIOS = np.array([0.5, 1.0, 2.0])
_STRIDES = [4, 8, 16, 32, 64]
_IMG = 512
_CH = 96
_A = 9
_MAX_RATIO = float(abs(math.log(1000.0 / 16.0)))
_CHUNK = 1024


def _anchor_wh(stride):
    h_ratios = np.sqrt(_ANCHOR_RATIOS)
    w_ratios = 1.0 / h_ratios
    ws = (stride * w_ratios[:, None] * _ANCHOR_SCALES[None, :]).reshape(-1)
    hs = (stride * h_ratios[:, None] * _ANCHOR_SCALES[None, :]).reshape(-1)
    return ws.astype(np.float32), hs.astype(np.float32)


def _conv_taps(t, xvs, wt_ref, ml, mr):
    for dyi in range(3):
        xv = xvs[dyi]
        for dxi in range(3):
            if dxi == 0:
                xs = jnp.roll(xv, 1, axis=1) * ml
            elif dxi == 2:
                xs = jnp.roll(xv, -1, axis=1) * mr
            else:
                xs = xv
            t = t + jnp.dot(wt_ref[dyi * 3 + dxi], xs,
                            preferred_element_type=jnp.float32)
    return t


def _decode_store(t, bt_ref, wr_ref, br_ref, ws, hs, cx, cy, out_ref):
    t = jnp.maximum(t + bt_ref[...], 0.0).astype(jnp.bfloat16)
    d = [jnp.dot(wr_ref[c], t, preferred_element_type=jnp.float32) + br_ref[c]
         for c in range(4)]
    pcx = d[0] * ws + cx
    pcy = d[1] * hs + cy
    pw = ws * jnp.exp(jnp.clip(d[2], -_MAX_RATIO, _MAX_RATIO))
    ph = hs * jnp.exp(jnp.clip(d[3], -_MAX_RATIO, _MAX_RATIO))
    out_ref[0, 0] = (pcx - 0.5 * pw).astype(jnp.bfloat16)
    out_ref[0, 1] = (pcy - 0.5 * ph).astype(jnp.bfloat16)
    out_ref[0, 2] = (pcx + 0.5 * pw).astype(jnp.bfloat16)
    out_ref[0, 3] = (pcy + 0.5 * ph).astype(jnp.bfloat16)


def _lvl0_kernel(x_ref, wt_ref, bt_ref, wr_ref, br_ref, anc_ref,
                 ml_ref, mr_ref, cx_ref, cy_ref, out_ref):
    j = pl.program_id(1)
    base = pl.multiple_of(j * _CHUNK, 128)
    xvs = [x_ref[0, :, pl.ds(base + 128 * k, _CHUNK)].astype(jnp.bfloat16)
           for k in range(3)]
    t = jnp.zeros((_CH, _CHUNK), dtype=jnp.float32)
    t = _conv_taps(t, xvs, wt_ref, ml_ref[...], mr_ref[...])
    _decode_store(t, bt_ref, wr_ref, br_ref, anc_ref[0], anc_ref[1],
                  cx_ref[...], cy_ref[...], out_ref)


def _rest_kernel(x_ref, wt_ref, bt_ref, wr_ref, br_ref,
                 ml_ref, mr_ref, cx_ref, cy_ref, wsl_ref, hsl_ref, out_ref):
    xvs = [x_ref[k, 0] for k in range(3)]
    t = jnp.zeros((_CH, _CHUNK), dtype=jnp.float32)
    t = _conv_taps(t, xvs, wt_ref, ml_ref[...], mr_ref[...])
    _decode_store(t, bt_ref, wr_ref, br_ref, wsl_ref[...], hsl_ref[...],
                  cx_ref[...], cy_ref[...], out_ref)


def _lane_arrays(widths, sizes, strides):
    mln, mrn, cxn, cyn = [], [], [], []
    for W, HW, s in zip(widths, sizes, strides):
        p = np.arange(HW)
        mln.append((p % W != 0).astype(np.float32))
        mrn.append((p % W != W - 1).astype(np.float32))
        cxn.append((p % W).astype(np.float32) * s)
        cyn.append((p // W).astype(np.float32) * s)
    return mln, mrn, cxn, cyn


def kernel(feat0, feat1, feat2, feat3, feat4, rpn_conv_w, rpn_conv_b,
           cls_w, cls_b, reg_w, reg_b):
    del cls_w, cls_b
    B = feat0.shape[0]

    w_taps = jnp.transpose(rpn_conv_w, (2, 3, 0, 1)).reshape(9, _CH, _CH)
    w_taps = w_taps.astype(jnp.bfloat16)
    bt = rpn_conv_b.reshape(_CH, 1)
    rw = reg_w.reshape(_A * 4, _CH)
    wregs = jnp.stack([rw[c::4] for c in range(4)]).astype(jnp.bfloat16)
    brs = jnp.stack([reg_b[c::4] for c in range(4)]).reshape(4, _A, 1)

    W0 = _IMG // _STRIDES[0]
    HW0 = W0 * W0
    nch0 = HW0 // _CHUNK
    x0 = jnp.pad(feat0.reshape(B, _CH, HW0), ((0, 0), (0, 0), (128, 128)))
    mln, mrn, cxn, cyn = _lane_arrays([W0], [HW0], _STRIDES[:1])
    ml0 = jnp.asarray(mln[0].reshape(1, HW0), dtype=jnp.bfloat16)
    mr0 = jnp.asarray(mrn[0].reshape(1, HW0), dtype=jnp.bfloat16)
    cx0 = jnp.asarray(cxn[0].reshape(1, HW0))
    cy0 = jnp.asarray(cyn[0].reshape(1, HW0))
    ws0, hs0 = _anchor_wh(_STRIDES[0])
    anc0 = jnp.asarray(np.stack([ws0, hs0]).reshape(2, _A, 1))

    out0 = pl.pallas_call(
        _lvl0_kernel,
        grid=(B, nch0),
        in_specs=[
            pl.BlockSpec((1, _CH, HW0 + 256), lambda b, j: (b, 0, 0)),
            pl.BlockSpec((9, _CH, _CH), lambda b, j: (0, 0, 0)),
            pl.BlockSpec((_CH, 1), lambda b, j: (0, 0)),
            pl.BlockSpec((4, _A, _CH), lambda b, j: (0, 0, 0)),
            pl.BlockSpec((4, _A, 1), lambda b, j: (0, 0, 0)),
            pl.BlockSpec((2, _A, 1), lambda b, j: (0, 0, 0)),
            pl.BlockSpec((1, _CHUNK), lambda b, j: (0, j)),
            pl.BlockSpec((1, _CHUNK), lambda b, j: (0, j)),
            pl.BlockSpec((1, _CHUNK), lambda b, j: (0, j)),
            pl.BlockSpec((1, _CHUNK), lambda b, j: (0, j)),
        ],
        out_specs=pl.BlockSpec((1, 4, _A, _CHUNK), lambda b, j: (b, 0, 0, j)),
        out_shape=jax.ShapeDtypeStruct((B, 4, _A, HW0), jnp.bfloat16),
    )(x0, w_taps, bt, wregs, brs, anc0, ml0, mr0, cx0, cy0)

    feats = [feat1, feat2, feat3, feat4]
    widths = [_IMG // s for s in _STRIDES[1:]]
    sizes = [w * w for w in widths]
    NV = sum(sizes)
    TOT = -(-NV // _CHUNK) * _CHUNK
    nch = TOT // _CHUNK

    shifted = {dy: [] for dy in (-1, 0, 1)}
    for f, W, HW in zip(feats, widths, sizes):
        xf = f.reshape(B, _CH, HW)
        xw = jnp.pad(xf, ((0, 0), (0, 0), (W, W)))
        for dy in (-1, 0, 1):
            shifted[dy].append(xw[:, :, W + dy * W: W + dy * W + HW])
    pad_tail = ((0, 0), (0, 0), (0, TOT - NV))
    xcat = jnp.stack([jnp.pad(jnp.concatenate(shifted[dy], axis=2), pad_tail)
                      for dy in (-1, 0, 1)]).astype(jnp.bfloat16)

    mln, mrn, cxn, cyn = _lane_arrays(widths, sizes, _STRIDES[1:])
    wsn, hsn = [], []
    for W, HW, s in zip(widths, sizes, _STRIDES[1:]):
        ws, hs = _anchor_wh(s)
        wsn.append(np.broadcast_to(ws[:, None], (_A, HW)))
        hsn.append(np.broadcast_to(hs[:, None], (_A, HW)))

    def _cat(parts, rows):
        a = np.concatenate(parts, axis=-1).reshape(rows, NV)
        return np.pad(a, ((0, 0), (0, TOT - NV)))

    ml = jnp.asarray(_cat(mln, 1), dtype=jnp.bfloat16)
    mr = jnp.asarray(_cat(mrn, 1), dtype=jnp.bfloat16)
    cx = jnp.asarray(_cat(cxn, 1))
    cy = jnp.asarray(_cat(cyn, 1))
    wsl = jnp.asarray(_cat(wsn, _A))
    hsl = jnp.asarray(_cat(hsn, _A))

    out1 = pl.pallas_call(
        _rest_kernel,
        grid=(B, nch),
        in_specs=[
            pl.BlockSpec((3, 1, _CH, _CHUNK), lambda b, j: (0, b, 0, j)),
            pl.BlockSpec((9, _CH, _CH), lambda b, j: (0, 0, 0)),
            pl.BlockSpec((_CH, 1), lambda b, j: (0, 0)),
            pl.BlockSpec((4, _A, _CH), lambda b, j: (0, 0, 0)),
            pl.BlockSpec((4, _A, 1), lambda b, j: (0, 0, 0)),
            pl.BlockSpec((1, _CHUNK), lambda b, j: (0, j)),
            pl.BlockSpec((1, _CHUNK), lambda b, j: (0, j)),
            pl.BlockSpec((1, _CHUNK), lambda b, j: (0, j)),
            pl.BlockSpec((1, _CHUNK), lambda b, j: (0, j)),
            pl.BlockSpec((_A, _CHUNK), lambda b, j: (0, j)),
            pl.BlockSpec((_A, _CHUNK), lambda b, j: (0, j)),
        ],
        out_specs=pl.BlockSpec((1, 4, _A, _CHUNK), lambda b, j: (b, 0, 0, j)),
        out_shape=jax.ShapeDtypeStruct((B, 4, _A, TOT), jnp.bfloat16),
    )(xcat, w_taps, bt, wregs, brs, ml, mr, cx, cy, wsl, hsl)

    def _finish(o, n):
        return jnp.transpose(o[:, :, :, :n],
                             (0, 3, 2, 1)).astype(jnp.float32).reshape(B, n * _A, 4)

    return jnp.concatenate([_finish(out0, HW0), _finish(out1, NV)], axis=1)


# speedup vs baseline: 1.0548x; 1.0548x over previous
import math

import jax
import jax.numpy as jnp
import numpy as np
from jax.experimental import pallas as pl

_ANCHOR_SCALES = np.array([8.0, 16.0, 32.0])
_ANCHOR_RATIOS = np.array([0.5, 1.0, 2.0])
_STRIDES = [4, 8, 16, 32, 64]
_IMG = 512
_CH = 96
_A = 9
_MAX_RATIO = float(abs(math.log(1000.0 / 16.0)))
_CHUNK = 2048


def _anchor_wh(stride):
    h_ratios = np.sqrt(_ANCHOR_RATIOS)
    w_ratios = 1.0 / h_ratios
    ws = (stride * w_ratios[:, None] * _ANCHOR_SCALES[None, :]).reshape(-1)
    hs = (stride * h_ratios[:, None] * _ANCHOR_SCALES[None, :]).reshape(-1)
    return ws.astype(np.float32), hs.astype(np.float32)


def _conv_taps(t, xvs, wt_ref, ml, mr):
    for dyi in range(3):
        xv = xvs[dyi]
        for dxi in range(3):
            if dxi == 0:
                xs = jnp.roll(xv, 1, axis=1) * ml
            elif dxi == 2:
                xs = jnp.roll(xv, -1, axis=1) * mr
            else:
                xs = xv
            t = t + jnp.dot(wt_ref[dyi * 3 + dxi], xs,
                            preferred_element_type=jnp.float32)
    return t


def _decode_store(t, bt_ref, wr_ref, br_ref, ws, hs, cx, cy, out_ref):
    t = jnp.maximum(t + bt_ref[...], 0.0).astype(jnp.bfloat16)
    d = [jnp.dot(wr_ref[c], t, preferred_element_type=jnp.float32) + br_ref[c]
         for c in range(4)]
    pcx = d[0] * ws + cx
    pcy = d[1] * hs + cy
    pw = ws * jnp.exp(jnp.clip(d[2], -_MAX_RATIO, _MAX_RATIO))
    ph = hs * jnp.exp(jnp.clip(d[3], -_MAX_RATIO, _MAX_RATIO))
    out_ref[0, 0] = (pcx - 0.5 * pw).astype(jnp.bfloat16)
    out_ref[0, 1] = (pcy - 0.5 * ph).astype(jnp.bfloat16)
    out_ref[0, 2] = (pcx + 0.5 * pw).astype(jnp.bfloat16)
    out_ref[0, 3] = (pcy + 0.5 * ph).astype(jnp.bfloat16)


def _lvl0_kernel(x_ref, wt_ref, bt_ref, wr_ref, br_ref, anc_ref,
                 ml_ref, mr_ref, cx_ref, cy_ref, out_ref):
    j = pl.program_id(1)
    base = pl.multiple_of(j * _CHUNK, 128)
    xvs = [x_ref[0, :, pl.ds(base + 128 * k, _CHUNK)].astype(jnp.bfloat16)
           for k in range(3)]
    t = jnp.zeros((_CH, _CHUNK), dtype=jnp.float32)
    t = _conv_taps(t, xvs, wt_ref, ml_ref[...], mr_ref[...])
    _decode_store(t, bt_ref, wr_ref, br_ref, anc_ref[0], anc_ref[1],
                  cx_ref[...], cy_ref[...], out_ref)


def _rest_kernel(x_ref, wt_ref, bt_ref, wr_ref, br_ref,
                 ml_ref, mr_ref, cx_ref, cy_ref, wsl_ref, hsl_ref, out_ref):
    xvs = [x_ref[k, 0] for k in range(3)]
    t = jnp.zeros((_CH, _CHUNK), dtype=jnp.float32)
    t = _conv_taps(t, xvs, wt_ref, ml_ref[...], mr_ref[...])
    _decode_store(t, bt_ref, wr_ref, br_ref, wsl_ref[...], hsl_ref[...],
                  cx_ref[...], cy_ref[...], out_ref)


def _lane_arrays(widths, sizes, strides):
    mln, mrn, cxn, cyn = [], [], [], []
    for W, HW, s in zip(widths, sizes, strides):
        p = np.arange(HW)
        mln.append((p % W != 0).astype(np.float32))
        mrn.append((p % W != W - 1).astype(np.float32))
        cxn.append((p % W).astype(np.float32) * s)
        cyn.append((p // W).astype(np.float32) * s)
    return mln, mrn, cxn, cyn


def kernel(feat0, feat1, feat2, feat3, feat4, rpn_conv_w, rpn_conv_b,
           cls_w, cls_b, reg_w, reg_b):
    del cls_w, cls_b
    B = feat0.shape[0]

    w_taps = jnp.transpose(rpn_conv_w, (2, 3, 0, 1)).reshape(9, _CH, _CH)
    w_taps = w_taps.astype(jnp.bfloat16)
    bt = rpn_conv_b.reshape(_CH, 1)
    rw = reg_w.reshape(_A * 4, _CH)
    wregs = jnp.stack([rw[c::4] for c in range(4)]).astype(jnp.bfloat16)
    brs = jnp.stack([reg_b[c::4] for c in range(4)]).reshape(4, _A, 1)

    W0 = _IMG // _STRIDES[0]
    HW0 = W0 * W0
    nch0 = HW0 // _CHUNK
    x0 = jnp.pad(feat0.reshape(B, _CH, HW0), ((0, 0), (0, 0), (128, 128)))
    mln, mrn, cxn, cyn = _lane_arrays([W0], [HW0], _STRIDES[:1])
    ml0 = jnp.asarray(mln[0].reshape(1, HW0), dtype=jnp.bfloat16)
    mr0 = jnp.asarray(mrn[0].reshape(1, HW0), dtype=jnp.bfloat16)
    cx0 = jnp.asarray(cxn[0].reshape(1, HW0))
    cy0 = jnp.asarray(cyn[0].reshape(1, HW0))
    ws0, hs0 = _anchor_wh(_STRIDES[0])
    anc0 = jnp.asarray(np.stack([ws0, hs0]).reshape(2, _A, 1))

    out0 = pl.pallas_call(
        _lvl0_kernel,
        grid=(B, nch0),
        in_specs=[
            pl.BlockSpec((1, _CH, HW0 + 256), lambda b, j: (b, 0, 0)),
            pl.BlockSpec((9, _CH, _CH), lambda b, j: (0, 0, 0)),
            pl.BlockSpec((_CH, 1), lambda b, j: (0, 0)),
            pl.BlockSpec((4, _A, _CH), lambda b, j: (0, 0, 0)),
            pl.BlockSpec((4, _A, 1), lambda b, j: (0, 0, 0)),
            pl.BlockSpec((2, _A, 1), lambda b, j: (0, 0, 0)),
            pl.BlockSpec((1, _CHUNK), lambda b, j: (0, j)),
            pl.BlockSpec((1, _CHUNK), lambda b, j: (0, j)),
            pl.BlockSpec((1, _CHUNK), lambda b, j: (0, j)),
            pl.BlockSpec((1, _CHUNK), lambda b, j: (0, j)),
        ],
        out_specs=pl.BlockSpec((1, 4, _A, _CHUNK), lambda b, j: (b, 0, 0, j)),
        out_shape=jax.ShapeDtypeStruct((B, 4, _A, HW0), jnp.bfloat16),
    )(x0, w_taps, bt, wregs, brs, anc0, ml0, mr0, cx0, cy0)

    feats = [feat1, feat2, feat3, feat4]
    widths = [_IMG // s for s in _STRIDES[1:]]
    sizes = [w * w for w in widths]
    NV = sum(sizes)
    TOT = -(-NV // _CHUNK) * _CHUNK
    nch = TOT // _CHUNK

    shifted = {dy: [] for dy in (-1, 0, 1)}
    for f, W, HW in zip(feats, widths, sizes):
        xf = f.reshape(B, _CH, HW)
        xw = jnp.pad(xf, ((0, 0), (0, 0), (W, W)))
        for dy in (-1, 0, 1):
            shifted[dy].append(xw[:, :, W + dy * W: W + dy * W + HW])
    pad_tail = ((0, 0), (0, 0), (0, TOT - NV))
    xcat = jnp.stack([jnp.pad(jnp.concatenate(shifted[dy], axis=2), pad_tail)
                      for dy in (-1, 0, 1)]).astype(jnp.bfloat16)

    mln, mrn, cxn, cyn = _lane_arrays(widths, sizes, _STRIDES[1:])
    wsn, hsn = [], []
    for W, HW, s in zip(widths, sizes, _STRIDES[1:]):
        ws, hs = _anchor_wh(s)
        wsn.append(np.broadcast_to(ws[:, None], (_A, HW)))
        hsn.append(np.broadcast_to(hs[:, None], (_A, HW)))

    def _cat(parts, rows):
        a = np.concatenate(parts, axis=-1).reshape(rows, NV)
        return np.pad(a, ((0, 0), (0, TOT - NV)))

    ml = jnp.asarray(_cat(mln, 1), dtype=jnp.bfloat16)
    mr = jnp.asarray(_cat(mrn, 1), dtype=jnp.bfloat16)
    cx = jnp.asarray(_cat(cxn, 1))
    cy = jnp.asarray(_cat(cyn, 1))
    wsl = jnp.asarray(_cat(wsn, _A))
    hsl = jnp.asarray(_cat(hsn, _A))

    out1 = pl.pallas_call(
        _rest_kernel,
        grid=(B, nch),
        in_specs=[
            pl.BlockSpec((3, 1, _CH, _CHUNK), lambda b, j: (0, b, 0, j)),
            pl.BlockSpec((9, _CH, _CH), lambda b, j: (0, 0, 0)),
            pl.BlockSpec((_CH, 1), lambda b, j: (0, 0)),
            pl.BlockSpec((4, _A, _CH), lambda b, j: (0, 0, 0)),
            pl.BlockSpec((4, _A, 1), lambda b, j: (0, 0, 0)),
            pl.BlockSpec((1, _CHUNK), lambda b, j: (0, j)),
            pl.BlockSpec((1, _CHUNK), lambda b, j: (0, j)),
            pl.BlockSpec((1, _CHUNK), lambda b, j: (0, j)),
            pl.BlockSpec((1, _CHUNK), lambda b, j: (0, j)),
            pl.BlockSpec((_A, _CHUNK), lambda b, j: (0, j)),
            pl.BlockSpec((_A, _CHUNK), lambda b, j: (0, j)),
        ],
        out_specs=pl.BlockSpec((1, 4, _A, _CHUNK), lambda b, j: (b, 0, 0, j)),
        out_shape=jax.ShapeDtypeStruct((B, 4, _A, TOT), jnp.bfloat16),
    )(xcat, w_taps, bt, wregs, brs, ml, mr, cx, cy, wsl, hsl)

    def _finish(o, n):
        return jnp.transpose(o[:, :, :, :n],
                             (0, 3, 2, 1)).astype(jnp.float32).reshape(B, n * _A, 4)

    return jnp.concatenate([_finish(out0, HW0), _finish(out1, NV)], axis=1)
